# Initial kernel scaffold; baseline (speedup 1.0000x reference)
import jax
import jax.numpy as jnp
from jax.experimental import pallas as pl


def kernel(preds_T, preds_S, im_ind, gt_T, gt_S, iter, gt_inds_T, gt_inds_S):
    raise NotImplementedError("write your pallas kernel here")


# speedup vs baseline: 3.2157x; 3.2157x over previous
import jax
import jax.numpy as jnp
from jax import lax
from jax.experimental import pallas as pl
from jax.experimental.pallas import tpu as pltpu
from jax.experimental.pallas import tpu_sc as plsc

K = 1000
KP = 1024
NT = 20000
NS = 5000
D = 784
RB = 1000
NBLK = NT // RB
NC = 2
NSUB = 16
NW = NC * NSUB
CH = 160
GC = 16
NCHUNK = CH // GC
EPS = 1e-5
BIG = 2**30


def _tc_body(gt_ref, x_ref, t_ref, minv_ref, rep_ref, runm_ref, runi_ref):
    i = pl.program_id(0)
    x = x_ref[...]
    t = t_ref[...]
    inter = jnp.sum(x * t, axis=1, keepdims=True)
    union = (jnp.sum(x * x, axis=1, keepdims=True)
             + jnp.sum(t * t, axis=1, keepdims=True) + EPS)
    loss = 1.0 - 2.0 * inter / union
    gt = gt_ref[0, 0, :]
    kiota = lax.broadcasted_iota(jnp.int32, (RB, KP), 1)
    masked = jnp.where(gt[:, None] == kiota, loss, jnp.inf)
    bmin = jnp.min(masked, axis=0, keepdims=True)
    riota = lax.broadcasted_iota(jnp.int32, (RB, KP), 0)
    bidx = jnp.min(jnp.where(masked == bmin, riota, BIG),
                   axis=0, keepdims=True)

    @pl.when(i == 0)
    def _():
        runm_ref[...] = jnp.full((1, KP), jnp.inf, jnp.float32)
        runi_ref[...] = jnp.zeros((1, KP), jnp.int32)

    upd = bmin < runm_ref[...]
    runm_ref[...] = jnp.where(upd, bmin, runm_ref[...])
    runi_ref[...] = jnp.where(upd, bidx + i * RB, runi_ref[...])

    @pl.when(i == NBLK - 1)
    def _():
        minv_ref[...] = runm_ref[...]
        rep_ref[...] = runi_ref[...]


def _tc_argmin(gt3, pT, gT):
    return pl.pallas_call(
        _tc_body,
        grid=(NBLK,),
        in_specs=[
            pl.BlockSpec((1, 1, RB), lambda i: (i, 0, 0)),
            pl.BlockSpec((RB, D), lambda i: (i, 0)),
            pl.BlockSpec((RB, D), lambda i: (i, 0)),
        ],
        out_specs=[
            pl.BlockSpec((1, KP), lambda i: (0, 0)),
            pl.BlockSpec((1, KP), lambda i: (0, 0)),
        ],
        out_shape=[
            jax.ShapeDtypeStruct((1, KP), jnp.float32),
            jax.ShapeDtypeStruct((1, KP), jnp.int32),
        ],
        scratch_shapes=[
            pltpu.VMEM((1, KP), jnp.float32),
            pltpu.VMEM((1, KP), jnp.int32),
        ],
    )(gt3, pT, gT)


def _vsum(v):
    parts = [v[i] for i in range(16)]
    while len(parts) > 1:
        parts = [parts[i] + parts[i + 1] for i in range(0, len(parts), 2)]
    return parts[0]


def _sc_body(minv_hbm, rep_hbm, gts_hbm, pT_hbm, pS_hbm, out_hbm,
             minv_v, rep_v, gs_v, ovec, tbuf, sbuf, sem_t, sem_s):
    wid = lax.axis_index("s") * NC + lax.axis_index("c")
    base = wid * CH
    pltpu.sync_copy(minv_hbm, minv_v.at[pl.ds(0, KP)])
    pltpu.sync_copy(rep_hbm, rep_v.at[pl.ds(0, KP)])
    pltpu.sync_copy(gts_hbm.at[pl.ds(base, CH)], gs_v)
    lanes = lax.iota(jnp.int32, 16)

    def chunk(c, total):
        gvec = gs_v[pl.ds(c * GC, GC)]
        rvec = jnp.zeros((GC,), jnp.int32)
        mvec = jnp.zeros((GC,), jnp.float32)
        for l in range(GC):
            g = gvec[l]
            rvec = jnp.where(lanes == l, rep_v[pl.ds(g, GC)][0], rvec)
            mvec = jnp.where(lanes == l, minv_v[pl.ds(g, GC)][0], mvec)
        jpos = base + c * GC + lanes
        valf = jnp.where((mvec < jnp.inf) & (jpos < NS),
                         jnp.float32(1.0), jnp.float32(0.0))
        srow = jnp.minimum(base + c * GC, NS - GC)
        cp_t = pltpu.async_copy(pT_hbm.at[rvec], tbuf, sem_t)
        cp_s = pltpu.async_copy(pS_hbm.at[pl.ds(srow, GC)], sbuf, sem_s)
        cp_t.wait()
        cp_s.wait()
        ivec = jnp.zeros((GC,), jnp.float32)
        uvec = jnp.ones((GC,), jnp.float32)
        for r in range(GC):
            def col(k, accs):
                aI, aX, aT = accs
                xv = sbuf[r, pl.ds(k * 16, 16)]
                tv = tbuf[r, pl.ds(k * 16, 16)]
                return (aI + xv * tv, aX + xv * xv, aT + tv * tv)
            z = jnp.zeros((16,), jnp.float32)
            aI, aX, aT = lax.fori_loop(0, D // 16, col, (z, z, z))
            ivec = jnp.where(lanes == r, _vsum(aI), ivec)
            uvec = jnp.where(lanes == r, _vsum(aX + aT) + EPS, uvec)
        pervec = 1.0 - 2.0 * ivec / uvec
        return total + valf * pervec

    total = lax.fori_loop(0, NCHUNK, chunk, jnp.zeros((16,), jnp.float32))
    ovec[...] = total
    pltpu.sync_copy(ovec, out_hbm.at[wid])


def _sc_call(minv, rep, gts_pad, pT, pS):
    mesh = plsc.VectorSubcoreMesh(core_axis_name="c", subcore_axis_name="s",
                                  num_cores=NC, num_subcores=NSUB)
    return pl.kernel(
        _sc_body,
        out_type=jax.ShapeDtypeStruct((NW, 16), jnp.float32),
        mesh=mesh,
        scratch_types=[
            pltpu.VMEM((KP + GC,), jnp.float32),
            pltpu.VMEM((KP + GC,), jnp.int32),
            pltpu.VMEM((CH,), jnp.int32),
            pltpu.VMEM((16,), jnp.float32),
            pltpu.VMEM((GC, D), jnp.float32),
            pltpu.VMEM((GC, D), jnp.float32),
            pltpu.SemaphoreType.DMA,
            pltpu.SemaphoreType.DMA,
        ],
        compiler_params=pltpu.CompilerParams(use_tc_tiling_on_sc=False),
    )(minv, rep, gts_pad, pT, pS)


def kernel(preds_T, preds_S, im_ind, gt_T, gt_S, iter, gt_inds_T, gt_inds_S):
    pT = preds_T.reshape(NT, D)
    gT = gt_T.reshape(NT, D)
    pS = preds_S.reshape(NS, D)
    gt3 = gt_inds_T.reshape(NBLK, 1, RB)
    minv, rep = _tc_argmin(gt3, pT, gT)
    gts_pad = jnp.concatenate(
        [gt_inds_S, jnp.zeros((NW * CH - NS,), gt_inds_S.dtype)])
    part = _sc_call(minv.reshape(KP), rep.reshape(KP), gts_pad, pT, pS)
    return jnp.sum(part)
